# baseline (speedup 1.0000x reference)
import functools

import jax
import jax.numpy as jnp
from jax import lax
from jax.experimental import pallas as pl
from jax.experimental.pallas import tpu as pltpu
from jax.experimental.pallas import tpu_sc as plsc

_K = 16
_HI = jax.lax.Precision.HIGHEST

_NC = 2
_NS = 16
_L = 16
_NW = _NC * _NS
_HALVES = 2
_NODES = (8 // _HALVES) * 256
_NPW = _NODES // _NW
_IPW = _NPW * _K
_WPB = 256 // _NPW
_CC = 128 // _L


def _tc_body(x_ref, wnT_ref, wcT_ref, b_ref, yn_ref, yc_ref, idx_ref, w_ref):
    x = x_ref[0]
    N = x.shape[0]
    G = jax.lax.dot_general(x, x, (((1,), (1,)), ((), ())), precision=_HI)
    sq_col = jnp.sum(x * x, axis=1, keepdims=True)
    ii = jax.lax.broadcasted_iota(jnp.int32, (N, N), 0)
    jj = jax.lax.broadcasted_iota(jnp.int32, (N, N), 1)
    eye = ii == jj
    sq_row = jnp.sum(jnp.where(eye, G, 0.0), axis=0, keepdims=True)

    yn_ref[0] = jnp.dot(x, wnT_ref[...], precision=_HI)
    yc_ref[0] = jnp.dot(x, wcT_ref[...], precision=_HI) + b_ref[...]

    neg = 2.0 * G - sq_col - sq_row
    inv_ri = 1.0 / jnp.sqrt(sq_row)
    big = jnp.int32(1 << 30)
    idx_rows = []
    w_rows = []
    for _ in range(_K):
        m = jnp.max(neg, axis=0, keepdims=True)
        ism = neg == m
        jsel = jnp.min(jnp.where(ism, ii, big), axis=0, keepdims=True)
        sq_j = jnp.sum(jnp.where(ism, sq_col, 0.0), axis=0, keepdims=True)
        g_sel = 0.5 * (m + sq_j + sq_row)
        w_rows.append(g_sel * inv_ri / jnp.sqrt(sq_j))
        idx_rows.append(jsel)
        neg = jnp.where(ism, -jnp.inf, neg)
    idx_ref[0] = jnp.concatenate(idx_rows, axis=0)
    w_ref[0] = jnp.concatenate(w_rows, axis=0)


def _sc_body(yn_hbm, yc_hbm, idx_hbm, w_hbm, out_hbm,
             idx_v, w_v, yc_v, out_v, ynl, sem):
    wid = lax.axis_index("s") * _NC + lax.axis_index("c")
    base = wid * _NPW
    batch = wid // _WPB
    h = pltpu.async_copy(yn_hbm.at[pl.ds(batch * 256, 256)], ynl, sem)
    pltpu.sync_copy(idx_hbm.at[pl.ds(base * _K, _IPW)], idx_v)
    pltpu.sync_copy(w_hbm.at[pl.ds(base * _K, _IPW)], w_v)
    pltpu.sync_copy(yc_hbm.at[pl.ds(base, _NPW)], yc_v)
    h.wait()

    def node_body(n, _):
        jvec = idx_v[pl.ds(n * _K, _K)]
        w_vec = w_v[pl.ds(n * _K, _K)]
        yc_vecs = [yc_v[n, pl.ds(cc * _L, _L)] for cc in range(_CC)]
        accs = [jnp.zeros((_L,), jnp.float32) for _ in range(_CC)]
        for k in range(_K):
            j = jvec[k]
            wk = w_vec[k]
            for cc in range(_CC):
                accs[cc] = jnp.maximum(
                    accs[cc],
                    (ynl[j, pl.ds(cc * _L, _L)] + yc_vecs[cc]) * wk)
        for cc in range(_CC):
            out_v[n, pl.ds(cc * _L, _L)] = accs[cc]
        return 0

    lax.fori_loop(0, _NPW, node_body, 0)
    pltpu.sync_copy(out_v, out_hbm.at[pl.ds(base, _NPW)])


@functools.partial(
    pl.kernel,
    out_type=jax.ShapeDtypeStruct((_NODES, 128), jnp.float32),
    mesh=plsc.VectorSubcoreMesh(core_axis_name="c", subcore_axis_name="s",
                                num_cores=_NC, num_subcores=_NS),
    scratch_types=[
        pltpu.VMEM((_IPW,), jnp.int32),
        pltpu.VMEM((_IPW,), jnp.float32),
        pltpu.VMEM((_NPW, 128), jnp.float32),
        pltpu.VMEM((_NPW, 128), jnp.float32),
        pltpu.VMEM((256, 128), jnp.float32),
        pltpu.SemaphoreType.DMA,
    ],
)
def _sc_agg(*args):
    _sc_body(*args)


def _tc_stage(xh, wnT, wcT, b2):
    Bh, N, C = xh.shape
    return pl.pallas_call(
        _tc_body,
        grid=(Bh,),
        in_specs=[
            pl.BlockSpec((1, N, C), lambda i: (i, 0, 0)),
            pl.BlockSpec((C, C), lambda i: (0, 0)),
            pl.BlockSpec((C, C), lambda i: (0, 0)),
            pl.BlockSpec((1, C), lambda i: (0, 0)),
        ],
        out_specs=[
            pl.BlockSpec((1, N, C), lambda i: (i, 0, 0)),
            pl.BlockSpec((1, N, C), lambda i: (i, 0, 0)),
            pl.BlockSpec((1, _K, N), lambda i: (i, 0, 0)),
            pl.BlockSpec((1, _K, N), lambda i: (i, 0, 0)),
        ],
        out_shape=[
            jax.ShapeDtypeStruct((Bh, N, C), jnp.float32),
            jax.ShapeDtypeStruct((Bh, N, C), jnp.float32),
            jax.ShapeDtypeStruct((Bh, _K, N), jnp.int32),
            jax.ShapeDtypeStruct((Bh, _K, N), jnp.float32),
        ],
    )(xh, wnT, wcT, b2)


def kernel(x, W, b):
    B, N, C = x.shape
    wnT = W[:, :C].T
    wcT = W[:, C:].T
    b2 = b.reshape(1, C)
    Bh = B // _HALVES
    outs = []
    for h in range(_HALVES):
        yn, yc, idx, w = _tc_stage(x[h * Bh:(h + 1) * Bh], wnT, wcT, b2)
        idx_nm = jnp.swapaxes(idx, 1, 2).reshape(-1)
        w_nm = jnp.swapaxes(w, 1, 2).reshape(-1)
        outs.append(_sc_agg(yn.reshape(_NODES, C), yc.reshape(_NODES, C),
                            idx_nm, w_nm))
    return jnp.concatenate(outs, axis=0).reshape(B, N, C)


# speedup vs baseline: 1.1159x; 1.1159x over previous
import functools

import jax
import jax.numpy as jnp
from jax import lax
from jax.experimental import pallas as pl
from jax.experimental.pallas import tpu as pltpu
from jax.experimental.pallas import tpu_sc as plsc

_K = 16
_HI = jax.lax.Precision.HIGHEST

_NC = 2
_NS = 16
_L = 16
_NW = _NC * _NS
_NODES = 8 * 256
_NPW = _NODES // _NW
_IPW = _NPW * _K
_CHUNK_ROWS = 128
_CHUNK_NODES = _CHUNK_ROWS // _K
_NCHUNKS = _NPW // _CHUNK_NODES
_CC = 128 // _L


def _tc_body(x_ref, wnT_ref, wcT_ref, b_ref, yn_ref, yc_ref, idx_ref, w_ref):
    x = x_ref[0]
    N = x.shape[0]
    G = jax.lax.dot_general(x, x, (((1,), (1,)), ((), ())), precision=_HI)
    sq_col = jnp.sum(x * x, axis=1, keepdims=True)
    ii = jax.lax.broadcasted_iota(jnp.int32, (N, N), 0)
    jj = jax.lax.broadcasted_iota(jnp.int32, (N, N), 1)
    eye = ii == jj
    sq_row = jnp.sum(jnp.where(eye, G, 0.0), axis=0, keepdims=True)

    yn_ref[0] = jnp.dot(x, wnT_ref[...], precision=_HI)
    yc_ref[0] = jnp.dot(x, wcT_ref[...], precision=_HI) + b_ref[...]

    neg = 2.0 * G - sq_col - sq_row
    inv_ri = 1.0 / jnp.sqrt(sq_row)
    big = jnp.int32(1 << 30)
    idx_rows = []
    w_rows = []
    for _ in range(_K):
        m = jnp.max(neg, axis=0, keepdims=True)
        ism = neg == m
        jsel = jnp.min(jnp.where(ism, ii, big), axis=0, keepdims=True)
        sq_j = jnp.sum(jnp.where(ism, sq_col, 0.0), axis=0, keepdims=True)
        g_sel = 0.5 * (m + sq_j + sq_row)
        w_rows.append(g_sel * inv_ri / jnp.sqrt(sq_j))
        idx_rows.append(jsel)
        neg = jnp.where(ism, -jnp.inf, neg)
    idx_ref[0] = jnp.concatenate(idx_rows, axis=0)
    w_ref[0] = jnp.concatenate(w_rows, axis=0)


def _sc_body(yn_hbm, yc_hbm, idx_hbm, w_hbm, out_hbm,
             idx_v, w_v, yc_v, out_v, ynl, sem, sem2):
    wid = lax.axis_index("s") * _NC + lax.axis_index("c")
    base = wid * _NPW
    batch = wid // (256 // _NPW)
    h = pltpu.async_copy(yn_hbm.at[pl.ds(batch * 256, 256)], ynl, sem)
    h1 = pltpu.async_copy(idx_hbm.at[pl.ds(base * _K, _IPW)], idx_v, sem2)
    h2 = pltpu.async_copy(w_hbm.at[pl.ds(base * _K, _IPW)], w_v, sem2)
    h3 = pltpu.async_copy(yc_hbm.at[pl.ds(base, _NPW)], yc_v, sem2)
    h1.wait()
    h2.wait()
    h3.wait()
    h.wait()

    def node_body(n, _):
        jvec = idx_v[pl.ds(n * _K, _K)]
        w_vec = w_v[pl.ds(n * _K, _K)]
        yc_vecs = [yc_v[n, pl.ds(cc * _L, _L)] for cc in range(_CC)]
        accs = [jnp.zeros((_L,), jnp.float32) for _ in range(_CC)]
        for k in range(_K):
            j = jvec[k]
            wk = w_vec[k]
            for cc in range(_CC):
                accs[cc] = jnp.maximum(
                    accs[cc],
                    (ynl[j, pl.ds(cc * _L, _L)] + yc_vecs[cc]) * wk)
        for cc in range(_CC):
            out_v[n, pl.ds(cc * _L, _L)] = accs[cc]
        return 0

    lax.fori_loop(0, _NPW, node_body, 0)
    pltpu.sync_copy(out_v, out_hbm.at[pl.ds(base, _NPW)])


@functools.partial(
    pl.kernel,
    out_type=jax.ShapeDtypeStruct((_NODES, 128), jnp.float32),
    mesh=plsc.VectorSubcoreMesh(core_axis_name="c", subcore_axis_name="s",
                                num_cores=_NC, num_subcores=_NS),
    scratch_types=[
        pltpu.VMEM((_IPW,), jnp.int32),
        pltpu.VMEM((_IPW,), jnp.float32),
        pltpu.VMEM((_NPW, 128), jnp.float32),
        pltpu.VMEM((_NPW, 128), jnp.float32),
        pltpu.VMEM((256, 128), jnp.float32),
        pltpu.SemaphoreType.DMA,
        pltpu.SemaphoreType.DMA,
    ],
)
def _sc_agg(*args):
    _sc_body(*args)


def kernel(x, W, b):
    B, N, C = x.shape
    wnT = W[:, :C].T
    wcT = W[:, C:].T
    b2 = b.reshape(1, C)
    yn, yc, idx, w = pl.pallas_call(
        _tc_body,
        grid=(B,),
        in_specs=[
            pl.BlockSpec((1, N, C), lambda i: (i, 0, 0)),
            pl.BlockSpec((C, C), lambda i: (0, 0)),
            pl.BlockSpec((C, C), lambda i: (0, 0)),
            pl.BlockSpec((1, C), lambda i: (0, 0)),
        ],
        out_specs=[
            pl.BlockSpec((1, N, C), lambda i: (i, 0, 0)),
            pl.BlockSpec((1, N, C), lambda i: (i, 0, 0)),
            pl.BlockSpec((1, _K, N), lambda i: (i, 0, 0)),
            pl.BlockSpec((1, _K, N), lambda i: (i, 0, 0)),
        ],
        out_shape=[
            jax.ShapeDtypeStruct((B, N, C), jnp.float32),
            jax.ShapeDtypeStruct((B, N, C), jnp.float32),
            jax.ShapeDtypeStruct((B, _K, N), jnp.int32),
            jax.ShapeDtypeStruct((B, _K, N), jnp.float32),
        ],
    )(x, wnT, wcT, b2)
    idx_nm = jnp.swapaxes(idx, 1, 2).reshape(-1)
    w_nm = jnp.swapaxes(w, 1, 2).reshape(-1)
    out = _sc_agg(yn.reshape(_NODES, C), yc.reshape(_NODES, C), idx_nm, w_nm)
    return out.reshape(B, N, C)


# speedup vs baseline: 1.1695x; 1.0481x over previous
import functools

import jax
import jax.numpy as jnp
from jax import lax
from jax.experimental import pallas as pl
from jax.experimental.pallas import tpu as pltpu
from jax.experimental.pallas import tpu_sc as plsc

_K = 16
_HI = jax.lax.Precision.HIGHEST

_NC = 2
_NS = 16
_L = 16
_NW = _NC * _NS
_NODES = 8 * 256
_NPW = _NODES // _NW
_IPW = _NPW * _K
_CHUNK_ROWS = 128
_CHUNK_NODES = _CHUNK_ROWS // _K
_NCHUNKS = _NPW // _CHUNK_NODES
_CC = 128 // _L


def _tc_body(x_ref, wnT_ref, wcT_ref, b_ref, yn_ref, yc_ref, idx_ref, w_ref):
    for bb in range(x_ref.shape[0]):
        _tc_batch(x_ref, wnT_ref, wcT_ref, b_ref, yn_ref, yc_ref, idx_ref,
                  w_ref, bb)


def _tc_batch(x_ref, wnT_ref, wcT_ref, b_ref, yn_ref, yc_ref, idx_ref, w_ref,
              bb):
    x = x_ref[bb]
    N = x.shape[0]
    G = jax.lax.dot_general(x, x, (((1,), (1,)), ((), ())), precision=_HI)
    sq_col = jnp.sum(x * x, axis=1, keepdims=True)
    ii = jax.lax.broadcasted_iota(jnp.int32, (N, N), 0)
    jj = jax.lax.broadcasted_iota(jnp.int32, (N, N), 1)
    eye = ii == jj
    sq_row = jnp.sum(jnp.where(eye, G, 0.0), axis=0, keepdims=True)

    yn_ref[bb] = jnp.dot(x, wnT_ref[...], precision=_HI)
    yc_ref[bb] = jnp.dot(x, wcT_ref[...], precision=_HI) + b_ref[...]

    neg = 2.0 * G - sq_col - sq_row
    inv_ri = 1.0 / jnp.sqrt(sq_row)
    big = jnp.int32(1 << 30)
    idx_rows = []
    w_rows = []
    for _ in range(_K):
        m = jnp.max(neg, axis=0, keepdims=True)
        ism = neg == m
        jsel = jnp.min(jnp.where(ism, ii, big), axis=0, keepdims=True)
        sq_j = jnp.sum(jnp.where(ism, sq_col, 0.0), axis=0, keepdims=True)
        g_sel = 0.5 * (m + sq_j + sq_row)
        w_rows.append(g_sel * inv_ri / jnp.sqrt(sq_j))
        idx_rows.append(jsel)
        neg = jnp.where(ism, -jnp.inf, neg)
    idx_ref[bb] = jnp.concatenate(idx_rows, axis=0)
    w_ref[bb] = jnp.concatenate(w_rows, axis=0)


def _sc_body(yn_hbm, yc_hbm, idx_hbm, w_hbm, out_hbm,
             idx_v, w_v, yc_v, out_v, ynl, sem, sem2):
    wid = lax.axis_index("s") * _NC + lax.axis_index("c")
    base = wid * _NPW
    batch = wid // (256 // _NPW)
    h = pltpu.async_copy(yn_hbm.at[pl.ds(batch * 256, 256)], ynl, sem)
    h1 = pltpu.async_copy(idx_hbm.at[pl.ds(base * _K, _IPW)], idx_v, sem2)
    h2 = pltpu.async_copy(w_hbm.at[pl.ds(base * _K, _IPW)], w_v, sem2)
    h3 = pltpu.async_copy(yc_hbm.at[pl.ds(base, _NPW)], yc_v, sem2)
    h1.wait()
    h2.wait()
    h3.wait()
    h.wait()

    @plsc.parallel_loop(0, _NPW, unroll=2)
    def node_body(n):
        jvec = idx_v[pl.ds(n * _K, _K)]
        w_vec = w_v[pl.ds(n * _K, _K)]
        yc_vecs = [yc_v[n, pl.ds(cc * _L, _L)] for cc in range(_CC)]
        accs = [jnp.zeros((_L,), jnp.float32) for _ in range(_CC)]
        for k in range(_K):
            j = jvec[k]
            wk = w_vec[k]
            for cc in range(_CC):
                accs[cc] = jnp.maximum(
                    accs[cc],
                    (ynl[j, pl.ds(cc * _L, _L)] + yc_vecs[cc]) * wk)
        for cc in range(_CC):
            out_v[n, pl.ds(cc * _L, _L)] = accs[cc]
    pltpu.sync_copy(out_v, out_hbm.at[pl.ds(base, _NPW)])


@functools.partial(
    pl.kernel,
    out_type=jax.ShapeDtypeStruct((_NODES, 128), jnp.float32),
    mesh=plsc.VectorSubcoreMesh(core_axis_name="c", subcore_axis_name="s",
                                num_cores=_NC, num_subcores=_NS),
    scratch_types=[
        pltpu.VMEM((_IPW,), jnp.int32),
        pltpu.VMEM((_IPW,), jnp.float32),
        pltpu.VMEM((_NPW, 128), jnp.float32),
        pltpu.VMEM((_NPW, 128), jnp.float32),
        pltpu.VMEM((256, 128), jnp.float32),
        pltpu.SemaphoreType.DMA,
        pltpu.SemaphoreType.DMA,
    ],
)
def _sc_agg(*args):
    _sc_body(*args)


def kernel(x, W, b):
    B, N, C = x.shape
    wnT = W[:, :C].T
    wcT = W[:, C:].T
    b2 = b.reshape(1, C)
    yn, yc, idx, w = pl.pallas_call(
        _tc_body,
        grid=(B // 2,),
        in_specs=[
            pl.BlockSpec((2, N, C), lambda i: (i, 0, 0)),
            pl.BlockSpec((C, C), lambda i: (0, 0)),
            pl.BlockSpec((C, C), lambda i: (0, 0)),
            pl.BlockSpec((1, C), lambda i: (0, 0)),
        ],
        out_specs=[
            pl.BlockSpec((2, N, C), lambda i: (i, 0, 0)),
            pl.BlockSpec((2, N, C), lambda i: (i, 0, 0)),
            pl.BlockSpec((2, _K, N), lambda i: (i, 0, 0)),
            pl.BlockSpec((2, _K, N), lambda i: (i, 0, 0)),
        ],
        out_shape=[
            jax.ShapeDtypeStruct((B, N, C), jnp.float32),
            jax.ShapeDtypeStruct((B, N, C), jnp.float32),
            jax.ShapeDtypeStruct((B, _K, N), jnp.int32),
            jax.ShapeDtypeStruct((B, _K, N), jnp.float32),
        ],
    )(x, wnT, wcT, b2)
    idx_nm = jnp.swapaxes(idx, 1, 2).reshape(-1)
    w_nm = jnp.swapaxes(w, 1, 2).reshape(-1)
    out = _sc_agg(yn.reshape(_NODES, C), yc.reshape(_NODES, C), idx_nm, w_nm)
    return out.reshape(B, N, C)
